# baseline (speedup 1.0000x reference)
import functools

import jax
import jax.numpy as jnp
from jax import lax
from jax.experimental import pallas as pl
from jax.experimental.pallas import tpu as pltpu
from jax.experimental.pallas import tpu_sc as plsc

B = 16384
NUM_DIM = 13
NUM_FIELDS = 26
VOCAB = 100000
EMB_DIM = 16
EPS = 1e-5

NT = 4
FPT = 8
GL = 128
IDX_ROWS = NT * B * FPT // GL

RB = 1024
NBLK = B // RB
H1 = 256
H2 = 128


def _sc_gather(tables_flat, idx_t):
  info = plsc.get_sparse_core_info()
  nw = info.num_cores * info.num_subcores
  bpw = B // nw
  irows_w = bpw * FPT // GL
  n_chunks = 2
  gpc = irows_w // n_chunks
  bpc = bpw // n_chunks
  mesh = plsc.VectorSubcoreMesh(core_axis_name="c", subcore_axis_name="s")
  eshape = jax.ShapeDtypeStruct((B, 128), jnp.float32)

  @functools.partial(
      pl.kernel,
      mesh=mesh,
      compiler_params=pltpu.CompilerParams(use_tc_tiling_on_sc=False),
      out_type=[eshape] * NT,
      scratch_types=[
          pltpu.VMEM((irows_w, GL), jnp.int32),
          pltpu.VMEM((gpc * GL, EMB_DIM), jnp.float32),
          pltpu.SemaphoreType.DMA,
      ],
  )
  def gather_kernel(tab_hbm, idx_hbm, e0, e1, e2, e3, idx_v, data_v, sem):
    wid = lax.axis_index("s") * info.num_cores + lax.axis_index("c")
    outs = [e0, e1, e2, e3]
    for t in range(NT):
      pltpu.sync_copy(
          idx_hbm.at[pl.ds(t * (IDX_ROWS // NT) + wid * irows_w, irows_w)],
          idx_v)
      for c in range(n_chunks):
        handles = []
        for j in range(gpc):
          handles.append(
              pltpu.async_copy(
                  tab_hbm.at[idx_v.at[c * gpc + j]],
                  data_v.at[pl.ds(j * GL, GL)], sem))
        for h in handles:
          h.wait()
        for v in range(FPT):
          pltpu.sync_copy(
              data_v.at[pl.ds(v * bpc, bpc)],
              outs[t].at[pl.ds(wid * bpw + c * bpc, bpc),
                         pl.ds(v * EMB_DIM, EMB_DIM)])

  return gather_kernel(tables_flat, idx_t)


def _mlp_body(xn_ref, e0_ref, e1_ref, e2_ref, e3_ref, w1a_ref, w1b0_ref,
              w1b1_ref, w1b2_ref, w1b3_ref, b1_ref, g1_ref, be1_ref, w2_ref,
              b2_ref, g2_ref, be2_ref, wpi_ref, bpi_ref, wmu_ref, bmu_ref,
              pi_ref, mu_ref, h1_s, h2_s, s1, q1, s2, q2):
  p = pl.program_id(0)
  i = pl.program_id(1)
  inv_b = 1.0 / B

  @pl.when(p == 0)
  def _phase0():
    @pl.when(i == 0)
    def _():
      s1[...] = jnp.zeros_like(s1)
      q1[...] = jnp.zeros_like(q1)

    h = (jnp.dot(xn_ref[...], w1a_ref[...], preferred_element_type=jnp.float32)
         + jnp.dot(e0_ref[...], w1b0_ref[...], preferred_element_type=jnp.float32)
         + jnp.dot(e1_ref[...], w1b1_ref[...], preferred_element_type=jnp.float32)
         + jnp.dot(e2_ref[...], w1b2_ref[...], preferred_element_type=jnp.float32)
         + jnp.dot(e3_ref[...], w1b3_ref[...], preferred_element_type=jnp.float32)
         + b1_ref[...])
    h1_s[pl.ds(i * RB, RB), :] = h
    s1[...] += jnp.sum(h, axis=0, keepdims=True)
    q1[...] += jnp.sum(h * h, axis=0, keepdims=True)

  @pl.when(p == 1)
  def _phase1():
    @pl.when(i == 0)
    def _():
      s2[...] = jnp.zeros_like(s2)
      q2[...] = jnp.zeros_like(q2)

    m = s1[...] * inv_b
    v = q1[...] * inv_b - m * m
    a = g1_ref[...] * lax.rsqrt(v + EPS)
    c = be1_ref[...] - m * a
    h = h1_s[pl.ds(i * RB, RB), :]
    hn = jnp.maximum(h * a + c, 0.0)
    h2 = jnp.dot(hn, w2_ref[...], preferred_element_type=jnp.float32) + b2_ref[...]
    h2_s[pl.ds(i * RB, RB), :] = h2
    s2[...] += jnp.sum(h2, axis=0, keepdims=True)
    q2[...] += jnp.sum(h2 * h2, axis=0, keepdims=True)

  @pl.when(p == 2)
  def _phase2():
    m = s2[...] * inv_b
    v = q2[...] * inv_b - m * m
    a = g2_ref[...] * lax.rsqrt(v + EPS)
    c = be2_ref[...] - m * a
    h = h2_s[pl.ds(i * RB, RB), :]
    hn = jnp.maximum(h * a + c, 0.0)
    logit = jnp.dot(hn, wpi_ref[...], preferred_element_type=jnp.float32) + bpi_ref[...]
    pi_ref[...] = jax.nn.sigmoid(logit)
    mu_ref[...] = jnp.dot(hn, wmu_ref[...], preferred_element_type=jnp.float32) + bmu_ref[...]


def _mlp(x_num, embs, w1a, w1bs, b1, g1, be1, w2, b2, g2, be2, wpi, bpi, wmu,
         bmu, interpret=False):
  def blk(p, i):
    return (jnp.where(p == 0, i, 0), 0)

  def const(p, i):
    return (0, 0)

  def out_blk(p, i):
    return (i, 0)

  grid = (3, NBLK)
  return pl.pallas_call(
      _mlp_body,
      grid=grid,
      in_specs=[pl.BlockSpec((RB, NUM_DIM), blk)]
      + [pl.BlockSpec((RB, 128), blk)] * NT
      + [pl.BlockSpec((NUM_DIM, H1), const)]
      + [pl.BlockSpec((128, H1), const)] * NT
      + [
          pl.BlockSpec((1, H1), const),
          pl.BlockSpec((1, H1), const),
          pl.BlockSpec((1, H1), const),
          pl.BlockSpec((H1, H2), const),
          pl.BlockSpec((1, H2), const),
          pl.BlockSpec((1, H2), const),
          pl.BlockSpec((1, H2), const),
          pl.BlockSpec((H2, 1), const),
          pl.BlockSpec((1, 1), const),
          pl.BlockSpec((H2, 1), const),
          pl.BlockSpec((1, 1), const),
      ],
      out_specs=[
          pl.BlockSpec((RB, 1), out_blk),
          pl.BlockSpec((RB, 1), out_blk),
      ],
      out_shape=[
          jax.ShapeDtypeStruct((B, 1), jnp.float32),
          jax.ShapeDtypeStruct((B, 1), jnp.float32),
      ],
      scratch_shapes=[
          pltpu.VMEM((B, H1), jnp.float32),
          pltpu.VMEM((B, H2), jnp.float32),
          pltpu.VMEM((1, H1), jnp.float32),
          pltpu.VMEM((1, H1), jnp.float32),
          pltpu.VMEM((1, H2), jnp.float32),
          pltpu.VMEM((1, H2), jnp.float32),
      ],
      compiler_params=pltpu.CompilerParams(
          dimension_semantics=("arbitrary", "arbitrary"),
          vmem_limit_bytes=100 * 1024 * 1024,
      ),
      interpret=interpret,
  )(x_num, *embs, w1a, *w1bs, b1, g1, be1, w2, b2, g2, be2, wpi, bpi, wmu, bmu)


def kernel(x_num, x_cat, tables, W1, b1, g1, be1, W2, b2, g2, be2, Wpi, bpi,
           Wmu, bmu):
  offs = jnp.concatenate([
      jnp.arange(NUM_FIELDS, dtype=jnp.int32) * VOCAB,
      jnp.zeros((NT * FPT - NUM_FIELDS,), jnp.int32),
  ])
  xpad = jnp.pad(x_cat, ((0, 0), (0, NT * FPT - NUM_FIELDS)))
  idx_t = (xpad + offs[None, :]).reshape(32, 2, 256, NT, FPT).transpose(
      3, 0, 1, 4, 2).reshape(IDX_ROWS, GL)
  tables_flat = tables.reshape(NUM_FIELDS * VOCAB, EMB_DIM)
  embs = _sc_gather(tables_flat, idx_t)

  w1a = W1[:NUM_DIM]
  w1b = jnp.zeros((NT * FPT * EMB_DIM, H1), W1.dtype).at[:NUM_FIELDS * EMB_DIM].set(
      W1[NUM_DIM:])
  w1bs = [w1b[t * 128:(t + 1) * 128] for t in range(NT)]
  pi, mu = _mlp(x_num, embs, w1a, w1bs, b1.reshape(1, H1), g1.reshape(1, H1),
                be1.reshape(1, H1), W2, b2.reshape(1, H2), g2.reshape(1, H2),
                be2.reshape(1, H2), Wpi, bpi.reshape(1, 1), Wmu,
                bmu.reshape(1, 1))
  return (pi, mu)
